# baseline (speedup 1.0000x reference)
import functools
import math

import jax
import jax.numpy as jnp
from jax import lax
from jax.experimental import pallas as pl
from jax.experimental.pallas import tpu as pltpu
from jax.experimental.pallas import tpu_sc as plsc

D_MODEL = 64
MAX_LEN = 50

NUM_CORES = 2
NUM_SUBCORES = 16
NW = NUM_CORES * NUM_SUBCORES
LANES = 16


def _build_sin_pos(max_len, d_model):
    position = jnp.arange(0, max_len, dtype=jnp.float32)[:, None]
    div_term = jnp.exp(
        jnp.arange(0, d_model, 2, dtype=jnp.float32)
        * (-math.log(10000.0) / d_model)
    )
    pe = jnp.zeros((max_len, d_model), dtype=jnp.float32)
    pe = pe.at[:, 0::2].set(jnp.sin(position * div_term))
    pe = pe.at[:, 1::2].set(jnp.cos(position * div_term))
    return pe


@functools.lru_cache(maxsize=None)
def _make_sc_kernel(N, D, T, chunk):
    assert N % NW == 0
    b_per_w = N // NW
    assert b_per_w % chunk == 0
    nchunk = b_per_w // chunk
    assert nchunk % 2 == 0
    assert chunk % T == 0 and b_per_w % T == 0
    assert D % LANES == 0
    dsub = D // LANES
    reps = chunk // T

    mesh = plsc.VectorSubcoreMesh(
        core_axis_name="c", subcore_axis_name="s",
        num_cores=NUM_CORES, num_subcores=NUM_SUBCORES)

    @functools.partial(
        pl.kernel,
        out_type=jax.ShapeDtypeStruct((N, D), jnp.float32),
        mesh=mesh,
        compiler_params=pltpu.CompilerParams(use_tc_tiling_on_sc=False),
        scratch_types=[
            pltpu.VMEM((b_per_w,), jnp.int32),
            pltpu.VMEM((chunk, D), jnp.float32),
            pltpu.VMEM((chunk, D), jnp.float32),
            pltpu.VMEM((T, D), jnp.float32),
            pltpu.SemaphoreType.DMA,
            pltpu.SemaphoreType.DMA,
            pltpu.SemaphoreType.DMA,
            pltpu.SemaphoreType.DMA,
        ],
    )
    def sc_kernel(table_hbm, idx_hbm, pos_hbm, out_hbm,
                  idx_v, rows0, rows1, pos_v,
                  gsem0, gsem1, wsem0, wsem1):
        wid = lax.axis_index("s") * NUM_CORES + lax.axis_index("c")
        base = wid * b_per_w
        rows = (rows0, rows1)
        gsems = (gsem0, gsem1)
        wsems = (wsem0, wsem1)

        pltpu.sync_copy(idx_hbm.at[pl.ds(base, b_per_w)], idx_v)
        pltpu.sync_copy(pos_hbm, pos_v)

        def issue_gather(c, b):
            pltpu.async_copy(
                table_hbm.at[idx_v.at[pl.ds(c * chunk, chunk)]],
                rows[b], gsems[b])

        def wait_gather(b):
            pltpu.make_async_copy(
                out_hbm.at[pl.ds(0, chunk)], rows[b], gsems[b]).wait()

        def issue_wb(c, b):
            pltpu.async_copy(
                rows[b], out_hbm.at[pl.ds(base + c * chunk, chunk)],
                wsems[b])

        def wait_wb(b):
            pltpu.make_async_copy(
                rows[b], out_hbm.at[pl.ds(0, chunk)], wsems[b]).wait()

        def add_pos(rows_ref):
            @pl.loop(0, T)
            def _(t):
                pvecs = [pos_v[t, pl.ds(d * LANES, LANES)]
                         for d in range(dsub)]
                for rep in range(reps):
                    r = t + T * rep
                    for d in range(dsub):
                        sl = pl.ds(d * LANES, LANES)
                        rows_ref[r, sl] += pvecs[d]

        issue_gather(0, 0)

        @pl.loop(0, nchunk // 2)
        def _(i):
            for b in range(2):
                c = i * 2 + b
                nb = 1 - b

                @pl.when(c > 0)
                def _():
                    wait_wb(nb)

                @pl.when(c + 1 < nchunk)
                def _():
                    issue_gather(c + 1, nb)

                wait_gather(b)
                add_pos(rows[b])
                issue_wb(c, b)

        wait_wb(1)

    return sc_kernel


def kernel(tokens, table):
    B, T = tokens.shape
    V, D = table.shape
    N = B * T
    idx = tokens.reshape(N).astype(jnp.int32)
    pos = _build_sin_pos(MAX_LEN, D)[:T, :]
    chunk = 8 * T
    sc = _make_sc_kernel(N, D, T, chunk)
    out = sc(table, idx, pos)
    return out.reshape(B, T, D)
